# Initial kernel scaffold; baseline (speedup 1.0000x reference)
import jax
import jax.numpy as jnp
from jax.experimental import pallas as pl


def kernel(signal, edge_index, edge_attr):
    raise NotImplementedError("write your pallas kernel here")


# speedup vs baseline: 167.7016x; 167.7016x over previous
import functools

import jax
import jax.numpy as jnp
from jax import lax
from jax.experimental import pallas as pl
from jax.experimental.pallas import tpu as pltpu
from jax.experimental.pallas import tpu_sc as plsc

N_NODES = 100000
N_EDGES = 3200000

NC = 2
NS = 16
L = 16

RW = 128
CHUNK_ROWS = 16
ROWS = N_EDGES // RW
ROWS_PER_W = ((ROWS + NC * NS - 1) // (NC * NS) + CHUNK_ROWS - 1) \
    // CHUNK_ROWS * CHUNK_ROWS
ROWS_PAD = ROWS_PER_W * NC * NS
NCHUNKS = ROWS_PER_W // CHUNK_ROWS

NPAD = 100352
PERT = NPAD // NS


def _sqrt16(x):
    i = plsc.bitcast(x, jnp.int32)
    y = plsc.bitcast((i >> 1) + jnp.int32(0x1FBD1DF5), jnp.float32)
    y = 0.5 * (y + x / y)
    y = 0.5 * (y + x / y)
    y = 0.5 * (y + x / y)
    return y


@functools.partial(
    pl.kernel,
    out_type=(
        jax.ShapeDtypeStruct((NC, NPAD), jnp.float32),
        jax.ShapeDtypeStruct((NC, NPAD), jnp.float32),
    ),
    mesh=plsc.VectorSubcoreMesh(core_axis_name="c", subcore_axis_name="s"),
    compiler_params=pltpu.CompilerParams(needs_layout_passes=False),
    scratch_types=[
        pltpu.VMEM((N_NODES,), jnp.float32),
        pltpu.VMEM((CHUNK_ROWS, RW), jnp.int32),
        pltpu.VMEM((CHUNK_ROWS, RW), jnp.int32),
        pltpu.VMEM((CHUNK_ROWS, RW), jnp.float32),
        pltpu.VMEM((CHUNK_ROWS, RW), jnp.float32),
        pltpu.VMEM((CHUNK_ROWS, RW), jnp.float32),
        pltpu.VMEM_SHARED((NPAD,), jnp.float32),
        pltpu.VMEM_SHARED((NPAD,), jnp.float32),
    ],
)
def _sc_scatter(sig_hbm, src_hbm, dst_hbm, attr_hbm, zeros_hbm,
                num_out, den_out,
                sigb, srcb, dstb, attrb, msgb, wb, num_sh, den_sh):
    cid = lax.axis_index("c")
    sid = lax.axis_index("s")
    wid = cid * NS + sid

    nslice = pl.ds(sid * PERT, PERT)
    pltpu.sync_copy(zeros_hbm.at[nslice], num_sh.at[nslice])
    pltpu.sync_copy(zeros_hbm.at[nslice], den_sh.at[nslice])
    pltpu.sync_copy(sig_hbm, sigb)
    plsc.subcore_barrier()

    row0 = wid * ROWS_PER_W

    def chunk_body(c, _):
        r0 = row0 + c * CHUNK_ROWS
        pltpu.sync_copy(src_hbm.at[pl.ds(r0, CHUNK_ROWS)], srcb)
        pltpu.sync_copy(dst_hbm.at[pl.ds(r0, CHUNK_ROWS)], dstb)
        pltpu.sync_copy(attr_hbm.at[pl.ds(r0, CHUNK_ROWS)], attrb)

        def row_body(j, _):
            for g in range(RW // L):
                sl = pl.ds(g * L, L)
                si = srcb[j, sl]
                di = dstb[j, sl]
                a = attrb[j, sl]
                xs = plsc.load_gather(sigb, [si])
                xd = plsc.load_gather(sigb, [di])
                w = _sqrt16(a)
                msgb[j, sl] = w * jnp.sign(xs - xd)
                wb[j, sl] = w
            return 0

        lax.fori_loop(0, CHUNK_ROWS, row_body, 0)

        def scat_body(j, _):
            pltpu.sync_copy(msgb.at[j], num_sh.at[dstb.at[j]], add=True)
            pltpu.sync_copy(wb.at[j], den_sh.at[dstb.at[j]], add=True)
            return 0

        lax.fori_loop(0, CHUNK_ROWS, scat_body, 0)
        return 0

    lax.fori_loop(0, NCHUNKS, chunk_body, 0)
    plsc.subcore_barrier()

    pltpu.sync_copy(num_sh.at[nslice], num_out.at[cid, nslice])
    pltpu.sync_copy(den_sh.at[nslice], den_out.at[cid, nslice])


def _combine(num_ref, den_ref, out_ref):
    num = num_ref[0] + num_ref[1]
    den = den_ref[0] + den_ref[1]
    out_ref[...] = num / den


def kernel(signal, edge_index, edge_attr):
    sig = signal.reshape(-1).astype(jnp.float32)
    src = edge_index[0].astype(jnp.int32)
    dst = edge_index[1].astype(jnp.int32)
    attr = edge_attr.reshape(-1).astype(jnp.float32)

    pad_e = ROWS_PAD * RW - N_EDGES
    src = jnp.concatenate([src, jnp.zeros((pad_e,), jnp.int32)])
    dst = jnp.concatenate([dst, jnp.full((pad_e,), N_NODES, jnp.int32)])
    attr = jnp.concatenate([attr, jnp.zeros((pad_e,), jnp.float32)])

    src2 = src.reshape(ROWS_PAD, RW)
    dst2 = dst.reshape(ROWS_PAD, RW)
    attr2 = attr.reshape(ROWS_PAD, RW)
    zeros = jnp.zeros((NPAD,), jnp.float32)

    num_parts, den_parts = _sc_scatter(sig, src2, dst2, attr2, zeros)

    out = pl.pallas_call(
        _combine,
        out_shape=jax.ShapeDtypeStruct((NPAD // RW, RW), jnp.float32),
    )(num_parts.reshape(NC, NPAD // RW, RW),
      den_parts.reshape(NC, NPAD // RW, RW))

    return out.reshape(NPAD)[:N_NODES].reshape(N_NODES, 1)
